# Initial kernel scaffold; baseline (speedup 1.0000x reference)
import jax
import jax.numpy as jnp
from jax.experimental import pallas as pl


def kernel(x_dict, edge_index, kw1, kb1, qw1, qb1, vw1, vb1, aw1, ab1, arel1, mrel1, prel1, skip1, kw2, kb2, qw2, qb2, vw2, vb2, aw2, ab2, arel2, mrel2, prel2, skip2, fcw, fcb):
    raise NotImplementedError("write your pallas kernel here")


# speedup vs baseline: 6.6755x; 6.6755x over previous
import math

import jax
import jax.numpy as jnp
from jax import lax
from jax.experimental import pallas as pl
from jax.experimental.pallas import tpu as pltpu
from jax.experimental.pallas import tpu_sc as plsc

N = 10000
E = 320000
D = 128
DH = D // 2
PW = DH + 16

NC = 2
NS = 16
NW = NC * NS
EW = E // NW
C = 80
NCH = EW // C
RPW = N // NS
ZR = 125

_F32 = jnp.float32


def _qkv_body(x_ref, kw_ref, qw_ref, vw_ref, arel_ref, mrel_ref,
              kb_ref, qb_ref, vb_ref, scale_ref, q_ref, k_ref,
              vlo_ref, vhi_ref):
    x = x_ref[...]
    kw_c = jnp.dot(kw_ref[...], arel_ref[...], preferred_element_type=_F32)
    vw_c = jnp.dot(vw_ref[...], mrel_ref[...], preferred_element_type=_F32)
    kb_c = jnp.dot(kb_ref[...], arel_ref[...], preferred_element_type=_F32)
    vb_c = jnp.dot(vb_ref[...], mrel_ref[...], preferred_element_type=_F32)
    scale = scale_ref[0, 0]
    q_ref[...] = (jnp.dot(x, qw_ref[...], preferred_element_type=_F32)
                  + qb_ref[...]) * scale
    k_ref[...] = jnp.dot(x, kw_c, preferred_element_type=_F32) + kb_c
    v = jnp.dot(x, vw_c, preferred_element_type=_F32) + vb_c
    vlo_ref[...] = v[:, :DH]
    vhi_ref[...] = v[:, DH:]


def _qkv(x, kw, qw, vw, arel, mrel, kb, qb, vb, scale):
    blk = 1000
    grid = (N // blk,)
    full = pl.BlockSpec((D, D), lambda i: (0, 0))
    row1 = pl.BlockSpec((1, D), lambda i: (0, 0))
    xb = pl.BlockSpec((blk, D), lambda i: (i, 0))
    hb = pl.BlockSpec((blk, DH), lambda i: (i, 0))
    return pl.pallas_call(
        _qkv_body,
        grid=grid,
        in_specs=[xb, full, full, full, full, full, row1, row1, row1,
                  pl.BlockSpec(memory_space=pltpu.SMEM)],
        out_specs=[xb, xb, hb, hb],
        out_shape=[jax.ShapeDtypeStruct((N, D), _F32),
                   jax.ShapeDtypeStruct((N, D), _F32),
                   jax.ShapeDtypeStruct((N, DH), _F32),
                   jax.ShapeDtypeStruct((N, DH), _F32)],
    )(x, kw, qw, vw, arel, mrel, kb, qb, vb, scale)


def _edge_body(q_hbm, k_hbm, vlo_hbm, vhi_hbm, src_hbm, dst_hbm, out_hbm,
               src_v, dst_v, qrow, krow, vrow, stage, zbuf, exs, acc, sem):
    c = lax.axis_index("c")
    s = lax.axis_index("s")
    wid = c * NS + s

    pltpu.sync_copy(src_hbm.at[wid], src_v)
    pltpu.sync_copy(dst_hbm.at[wid], dst_v)

    zv = jnp.zeros((16,), _F32)
    ii = lax.iota(jnp.int32, 16)
    oneh = jnp.where(ii == 0, _F32(1.0), _F32(0.0))

    def zrow(r, carry):
        def zcol(j, carry2):
            zbuf[r, pl.ds(j * 16, 16)] = zv
            return carry2
        return lax.fori_loop(0, PW // 16, zcol, carry)
    lax.fori_loop(0, ZR, zrow, 0)

    base_r = s * RPW

    def zero_acc():
        for i in range(RPW // ZR):
            pltpu.sync_copy(zbuf, acc.at[pl.ds(base_r + i * ZR, ZR)])

    zero_acc()
    plsc.subcore_barrier()

    def chunk1(ci, carry):
        sidx = src_v.at[ci]
        didx = dst_v.at[ci]
        cp_q = pltpu.async_copy(q_hbm.at[didx], qrow, sem)
        cp_k = pltpu.async_copy(k_hbm.at[sidx], krow, sem)
        cp_v = pltpu.async_copy(vlo_hbm.at[sidx], vrow, sem)
        cp_q.wait()
        cp_k.wait()
        cp_v.wait()

        def edot(t, carry2):
            alpha_vec = zv
            for l in range(16):
                e = t * 16 + l
                acc16 = qrow[e, pl.ds(0, 16)] * krow[e, pl.ds(0, 16)]
                for j in range(1, D // 16):
                    acc16 = acc16 + (qrow[e, pl.ds(j * 16, 16)]
                                     * krow[e, pl.ds(j * 16, 16)])
                alpha_vec = jnp.where(ii == l, jnp.sum(acc16), alpha_vec)
            exv = jnp.exp(alpha_vec)
            exs[pl.ds(ci * C + t * 16, 16)] = exv
            for l in range(16):
                e = t * 16 + l
                b = lax.broadcast_in_dim(exv[l], (16,), ())
                for j in range(DH // 16):
                    stage[e, pl.ds(j * 16, 16)] = (vrow[e, pl.ds(j * 16, 16)]
                                                   * b)
                stage[e, pl.ds(DH, 16)] = b * oneh
            return carry2
        lax.fori_loop(0, C // 16, edot, 0)

        pltpu.sync_copy(stage, acc.at[didx], add=True)
        return carry
    lax.fori_loop(0, NCH, chunk1, 0)

    plsc.subcore_barrier()
    pltpu.sync_copy(acc.at[pl.ds(base_r, RPW)],
                    out_hbm.at[c, 0, pl.ds(base_r, RPW)])
    zero_acc()
    plsc.subcore_barrier()

    def chunk2(ci, carry):
        sidx = src_v.at[ci]
        didx = dst_v.at[ci]
        pltpu.async_copy(vhi_hbm.at[sidx], vrow, sem).wait()

        def escale(t, carry2):
            exv = exs[pl.ds(ci * C + t * 16, 16)]
            for l in range(16):
                e = t * 16 + l
                b = lax.broadcast_in_dim(exv[l], (16,), ())
                for j in range(DH // 16):
                    stage[e, pl.ds(j * 16, 16)] = (vrow[e, pl.ds(j * 16, 16)]
                                                   * b)
            return carry2
        lax.fori_loop(0, C // 16, escale, 0)

        pltpu.sync_copy(stage, acc.at[didx], add=True)
        return carry
    lax.fori_loop(0, NCH, chunk2, 0)

    plsc.subcore_barrier()
    pltpu.sync_copy(acc.at[pl.ds(base_r, RPW)],
                    out_hbm.at[c, 1, pl.ds(base_r, RPW)])


def _edge(q, k, vlo, vhi, src3, dst3):
    mesh = plsc.VectorSubcoreMesh(core_axis_name="c", subcore_axis_name="s")
    fn = pl.kernel(
        _edge_body,
        out_type=jax.ShapeDtypeStruct((NC, 2, N, PW), _F32),
        mesh=mesh,
        compiler_params=pltpu.CompilerParams(use_tc_tiling_on_sc=False,
                                             needs_layout_passes=False),
        scratch_types=[
            pltpu.VMEM((NCH, C), jnp.int32),
            pltpu.VMEM((NCH, C), jnp.int32),
            pltpu.VMEM((C, D), _F32),
            pltpu.VMEM((C, D), _F32),
            pltpu.VMEM((C, DH), _F32),
            pltpu.VMEM((C, PW), _F32),
            pltpu.VMEM((ZR, PW), _F32),
            pltpu.VMEM((EW,), _F32),
            pltpu.VMEM_SHARED((N, PW), _F32),
            pltpu.SemaphoreType.DMA,
        ],
    )
    return fn(q, k, vlo, vhi, src3, dst3)


def _agg_from_partials(p):
    s = p[0] + p[1]
    num = jnp.concatenate([s[0, :, :DH], s[1, :, :DH]], axis=1)
    den = s[0, :, DH:DH + 1]
    return num / (den + 1e-16)


def _post_body(x_ref, p_ref, aw_ref, ab_ref, sk_ref, o_ref):
    agg = _agg_from_partials(p_ref[...])
    out = jax.nn.gelu(agg) @ aw_ref[...] + ab_ref[...]
    sk = sk_ref[0, 0]
    h = sk * out + (1.0 - sk) * x_ref[...]
    o_ref[...] = jnp.maximum(h, 0.0)


def _post_final_body(x_ref, p_ref, aw_ref, ab_ref, sk_ref,
                     fcw_ref, fcb_ref, o_ref):
    agg = _agg_from_partials(p_ref[...])
    out = jax.nn.gelu(agg) @ aw_ref[...] + ab_ref[...]
    sk = sk_ref[0, 0]
    h = sk * out + (1.0 - sk) * x_ref[...]
    h = jnp.maximum(h, 0.0)
    o_ref[...] = jax.nn.sigmoid(
        jnp.dot(h, fcw_ref[...], preferred_element_type=_F32) + fcb_ref[...])


def _post(x, p, aw, ab, sk):
    blk = 1000
    xb = pl.BlockSpec((blk, D), lambda i: (i, 0))
    return pl.pallas_call(
        _post_body,
        grid=(N // blk,),
        in_specs=[xb,
                  pl.BlockSpec((NC, 2, blk, PW), lambda i: (0, 0, i, 0)),
                  pl.BlockSpec((D, D), lambda i: (0, 0)),
                  pl.BlockSpec((1, D), lambda i: (0, 0)),
                  pl.BlockSpec(memory_space=pltpu.SMEM)],
        out_specs=xb,
        out_shape=jax.ShapeDtypeStruct((N, D), _F32),
    )(x, p, aw, ab, sk)


def _post_final(x, p, aw, ab, sk, fcw, fcb):
    blk = 1000
    xb = pl.BlockSpec((blk, D), lambda i: (i, 0))
    return pl.pallas_call(
        _post_final_body,
        grid=(N // blk,),
        in_specs=[xb,
                  pl.BlockSpec((NC, 2, blk, PW), lambda i: (0, 0, i, 0)),
                  pl.BlockSpec((D, D), lambda i: (0, 0)),
                  pl.BlockSpec((1, D), lambda i: (0, 0)),
                  pl.BlockSpec(memory_space=pltpu.SMEM),
                  pl.BlockSpec((D, 1), lambda i: (0, 0)),
                  pl.BlockSpec((1, 1), lambda i: (0, 0))],
        out_specs=pl.BlockSpec((blk, 1), lambda i: (i, 0)),
        out_shape=jax.ShapeDtypeStruct((N, 1), _F32),
    )(x, p, aw, ab, sk, fcw, fcb)


def kernel(x_dict, edge_index, kw1, kb1, qw1, qb1, vw1, vb1, aw1, ab1,
           arel1, mrel1, prel1, skip1, kw2, kb2, qw2, qb2, vw2, vb2, aw2,
           ab2, arel2, mrel2, prel2, skip2, fcw, fcb):
    src3 = edge_index[0].reshape(NW, NCH, C)
    dst3 = edge_index[1].reshape(NW, NCH, C)
    inv_sqrt_d = 1.0 / math.sqrt(D)

    def layer(x, kw, kb, qw, qb, vw, vb, aw, ab, arel, mrel, prel, skip,
              final):
        scale = (prel * inv_sqrt_d).astype(_F32).reshape(1, 1)
        sk = jax.nn.sigmoid(skip).astype(_F32).reshape(1, 1)
        q, k, vlo, vhi = _qkv(x, kw, qw, vw, arel, mrel,
                              kb.reshape(1, D), qb.reshape(1, D),
                              vb.reshape(1, D), scale)
        p = _edge(q, k, vlo, vhi, src3, dst3)
        if final:
            return _post_final(x, p, aw, ab.reshape(1, D), sk,
                               fcw, fcb.reshape(1, 1))
        return _post(x, p, aw, ab.reshape(1, D), sk)

    h = layer(x_dict, kw1, kb1, qw1, qb1, vw1, vb1, aw1, ab1, arel1, mrel1,
              prel1, skip1, final=False)
    return layer(h, kw2, kb2, qw2, qb2, vw2, vb2, aw2, ab2, arel2, mrel2,
                 prel2, skip2, final=True)
